# baseline (speedup 1.0000x reference)
import numpy as np
import jax
import jax.numpy as jnp
from jax import lax
from jax.experimental import pallas as pl
from jax.experimental.pallas import tpu as pltpu
from jax.experimental.pallas import tpu_sc as plsc

_SHAPE = (160, 160, 160)
_rng = np.random.default_rng(0)
_K = []
for _s in _SHAPE:
    _kk = _rng.normal(32.0, 16.0)
    _K.append(int(np.clip(np.floor(_kk), 4, _s)))
_K = tuple(_K)
_N = tuple((s + (k - s % k)) // k for s, k in zip(_SHAPE, _K))
_NB = _N[0] * _N[1] * _N[2]
_PERM = _rng.permutation(_NB)

_B = 2
_NW = 32
_K0, _K1, _K2 = _K
_N0, _N1, _N2 = _N
_C0 = tuple(ix * _K2 for ix in range(_N2))


def _unflatten(i):
    return i // (_N1 * _N2), (i // _N2) % _N1, i % _N2


def _build_params():
    tasks = []
    for b in range(_B):
        for iz in range(_N0):
            eo0 = min(_K0, _SHAPE[0] - iz * _K0)
            for iy in range(_N1):
                eo1 = min(_K1, _SHAPE[1] - iy * _K1)
                for oz in range(eo0):
                    p = [0] * 32
                    p[0] = b * _SHAPE[0] + iz * _K0 + oz
                    p[1] = iy * _K1
                    p[2] = eo1
                    for ix in range(_N2):
                        i = iz * (_N1 * _N2) + iy * _N2 + ix
                        jz, jy, jx = _unflatten(int(_PERM[i]))
                        es0 = min(_K0, _SHAPE[0] - jz * _K0)
                        es1 = min(_K1, _SHAPE[1] - jy * _K1)
                        es2 = min(_K2, _SHAPE[2] - jx * _K2)
                        eo2 = min(_K2, _SHAPE[2] - ix * _K2)
                        m1 = min(es1, eo1)
                        m2 = min(es2, eo2)
                        sx = jx * _K2
                        zv = oz < es0
                        sy = jy * _K1
                        syc = min(sy, _SHAPE[1] - 29)
                        base = 4 + ix * 6
                        p[base + 0] = (b * _SHAPE[0] + jz * _K0 + oz) if zv else 0
                        p[base + 1] = syc if zv else 0
                        p[base + 2] = (sy - syc) if zv else 0
                        p[base + 3] = sx
                        p[base + 4] = m1 if zv else 0
                        flags = (1 if m2 == _K2 else 0)
                        flags |= (2 if (m2 < _K2 and eo2 == _K2) else 0)
                        p[base + 5] = flags
                    tasks.append(p)
    assert len(tasks) == _NW * 60
    rows = np.zeros((_NW, 61, 32), dtype=np.int32)
    for t, p in enumerate(tasks):
        rows[t % _NW, t // _NW] = p
    return rows


_PARAMS = _build_params()

_CHUNK_SRC = {
    0: [0], 1: [0], 2: [0, 1], 3: [1], 4: [1],
    5: [1, 2], 6: [2], 7: [2, 3], 8: [3], 9: [3],
}
_SPLIT = {2: 10, 5: 4, 7: 14}
_ZT_LANE = {(2, 0): 2, (4, 1): 12, (5, 1): 0, (7, 2): 6}


def _body(x_hbm, prm_hbm, out_hbm, prm_v, pnl_v, out_v, sem_r, sem_w):
    wid = lax.axis_index("s") * 2 + lax.axis_index("c")
    pltpu.sync_copy(prm_hbm.at[wid], prm_v)
    lanesi = lax.iota(jnp.int32, 16)
    zero_i = lanesi * 0
    ixvecs = [zero_i + ix for ix in range(_N2)]
    lanesf = lanesi.astype(jnp.float32)
    onev = jnp.clip(lanesf + 1.0, 0.0, 1.0)
    mcv = {q: jnp.clip(float(s) - lanesf, 0.0, 1.0)
           for q, s in _SPLIT.items()}
    ztcv = {k: jnp.clip(lanesf - float(zl - 1), 0.0, 1.0)
            for k, zl in _ZT_LANE.items()}

    def load_scalars(s):
        rowA = prm_v[s, pl.ds(0, 16)]
        rowB = prm_v[s, pl.ds(16, 16)]
        vals = [rowA[0], rowA[1], rowA[2]]
        for ix in range(_N2):
            base = 4 + ix * 6
            src = rowA if base + 5 < 16 else rowB
            off = base if base + 5 < 16 else base - 16
            vals += [src[off + 0], src[off + 1], src[off + 2],
                     src[off + 3], src[off + 4], src[off + 5]]
        return tuple(vals)

    def sc_view(vals):
        sc = {"dzg": vals[0], "dy": vals[1], "eo1": vals[2],
              "szg": [], "sy": [], "dsy": [], "av": [], "m1eff": [],
              "flags": []}
        for ix in range(_N2):
            b = 3 + ix * 6
            sc["szg"].append(vals[b + 0])
            sc["sy"].append(vals[b + 1])
            sc["dsy"].append(vals[b + 2])
            sc["av"].append(vals[b + 3])
            sc["m1eff"].append(vals[b + 4])
            sc["flags"].append(vals[b + 5])
        return sc

    def reads(sc, slot, start):
        for ix in range(_N2):
            cp = pltpu.make_async_copy(
                x_hbm.at[sc["szg"][ix], pl.ds(sc["sy"][ix], 29), :],
                pnl_v.at[slot, ix],
                sem_r.at[slot])
            if start:
                cp.start()
            else:
                cp.wait()

    def write_start(sc, slot):
        for E1 in (29, 15):
            @pl.when(sc["eo1"] == E1)
            def _(E1=E1):
                pltpu.make_async_copy(
                    out_v.at[slot, pl.ds(0, E1), :],
                    out_hbm.at[sc["dzg"], pl.ds(sc["dy"], E1), :],
                    sem_w.at[slot]).start()

    def write_wait(w, slot):
        dzg, dy, eo1 = w
        for E1 in (29, 15):
            @pl.when(eo1 == E1)
            def _(E1=E1):
                pltpu.make_async_copy(
                    out_v.at[slot, pl.ds(0, E1), :],
                    out_hbm.at[dzg, pl.ds(dy, E1), :],
                    sem_w.at[slot]).wait()

    def assemble(sc, slot):
        ztmul = {}
        for (q, ix), c in ztcv.items():
            zt_f = jnp.where((sc["flags"][ix] & 2) != 0, 1.0, 0.0)
            ztmul[(q, ix)] = onev - zt_f * c
        slot_vec = zero_i + slot
        dsyvec = [zero_i + sc["dsy"][ix] for ix in range(_N2)]

        @plsc.parallel_loop(0, sc["eo1"], unroll=4)
        def row_body(r):
            valid_f = [jnp.where(r < sc["m1eff"][ix], 1.0, 0.0)
                       for ix in range(_N2)]
            rvec = zero_i + r
            for q in range(10):
                pieces = []
                for ix in _CHUNK_SRC[q]:
                    off = sc["av"][ix] + (16 * q - _C0[ix])
                    v = plsc.load_gather(
                        pnl_v, [slot_vec, ixvecs[ix], rvec + dsyvec[ix],
                                lanesi + off])
                    v = v * valid_f[ix]
                    if (q, ix) in ztmul:
                        v = v * ztmul[(q, ix)]
                    pieces.append(v)
                if len(pieces) == 2:
                    chunk = pieces[1] + (pieces[0] - pieces[1]) * mcv[q]
                else:
                    chunk = pieces[0]
                out_v[slot, r, pl.ds(16 * q, 16)] = chunk

    sc0_t = load_scalars(0)
    reads(sc_view(sc0_t), 0, True)
    dummy_w = (sc0_t[0] * 0, sc0_t[0] * 0, sc0_t[0] * 0 + 29)

    def slice_body(s, carry):
        sc_t, w1, w2 = carry
        slot = lax.rem(s, 2)
        sc = sc_view(sc_t)
        scn_t = load_scalars(s + 1)

        @pl.when(s < 59)
        def _():
            reads(sc_view(scn_t), 1 - slot, True)

        reads(sc, slot, False)

        @pl.when(s >= 2)
        def _():
            write_wait(w2, slot)

        assemble(sc, slot)
        write_start(sc, slot)
        return (scn_t, (sc["dzg"], sc["dy"], sc["eo1"]), w1)

    fin = lax.fori_loop(0, 60, slice_body, (sc0_t, dummy_w, dummy_w))
    write_wait(fin[2], 0)
    write_wait(fin[1], 1)


_run = pl.kernel(
    _body,
    out_type=jax.ShapeDtypeStruct((_B * _SHAPE[0], _SHAPE[1], _SHAPE[2]),
                                  jnp.float32),
    mesh=plsc.VectorSubcoreMesh(core_axis_name="c", subcore_axis_name="s"),
    compiler_params=pltpu.CompilerParams(use_tc_tiling_on_sc=False,
                                         needs_layout_passes=False),
    scratch_types=[
        pltpu.VMEM((61, 32), jnp.int32),
        pltpu.VMEM((2, 4, 29, 160), jnp.float32),
        pltpu.VMEM((2, 29, 160), jnp.float32),
        pltpu.SemaphoreType.DMA((2,)),
        pltpu.SemaphoreType.DMA((2,)),
    ],
)


def kernel(x):
    xf = x.reshape(_B * _SHAPE[0], _SHAPE[1], _SHAPE[2])
    prm = jnp.asarray(_PARAMS)
    out = _run(xf, prm)
    return out.reshape(_B, 1, *_SHAPE)


# speedup vs baseline: 1.0035x; 1.0035x over previous
import numpy as np
import jax
import jax.numpy as jnp
from jax import lax
from jax.experimental import pallas as pl
from jax.experimental.pallas import tpu as pltpu
from jax.experimental.pallas import tpu_sc as plsc

_SHAPE = (160, 160, 160)
_rng = np.random.default_rng(0)
_K = []
for _s in _SHAPE:
    _kk = _rng.normal(32.0, 16.0)
    _K.append(int(np.clip(np.floor(_kk), 4, _s)))
_K = tuple(_K)
_N = tuple((s + (k - s % k)) // k for s, k in zip(_SHAPE, _K))
_NB = _N[0] * _N[1] * _N[2]
_PERM = _rng.permutation(_NB)

_B = 2
_NW = 32
_K0, _K1, _K2 = _K
_N0, _N1, _N2 = _N
_C0 = tuple(ix * _K2 for ix in range(_N2))


def _unflatten(i):
    return i // (_N1 * _N2), (i // _N2) % _N1, i % _N2


def _build_params():
    tasks = []
    for b in range(_B):
        for iz in range(_N0):
            eo0 = min(_K0, _SHAPE[0] - iz * _K0)
            for iy in range(_N1):
                eo1 = min(_K1, _SHAPE[1] - iy * _K1)
                for oz in range(eo0):
                    p = [0] * 32
                    p[0] = b * _SHAPE[0] + iz * _K0 + oz
                    p[1] = iy * _K1
                    p[2] = eo1
                    for ix in range(_N2):
                        i = iz * (_N1 * _N2) + iy * _N2 + ix
                        jz, jy, jx = _unflatten(int(_PERM[i]))
                        es0 = min(_K0, _SHAPE[0] - jz * _K0)
                        es1 = min(_K1, _SHAPE[1] - jy * _K1)
                        es2 = min(_K2, _SHAPE[2] - jx * _K2)
                        eo2 = min(_K2, _SHAPE[2] - ix * _K2)
                        m1 = min(es1, eo1)
                        m2 = min(es2, eo2)
                        sx = jx * _K2
                        zv = oz < es0
                        sy = jy * _K1
                        syc = min(sy, _SHAPE[1] - 29)
                        base = 4 + ix * 6
                        p[base + 0] = (b * _SHAPE[0] + jz * _K0 + oz) if zv else 0
                        p[base + 1] = syc if zv else 0
                        p[base + 2] = (sy - syc) if zv else 0
                        p[base + 3] = sx
                        p[base + 4] = m1 if zv else 0
                        flags = (1 if m2 == _K2 else 0)
                        flags |= (2 if (m2 < _K2 and eo2 == _K2) else 0)
                        p[base + 5] = flags
                    tasks.append(p)
    assert len(tasks) == _NW * 60
    rows = np.zeros((_NW, 61, 32), dtype=np.int32)
    for t, p in enumerate(tasks):
        rows[t % _NW, t // _NW] = p
    return rows


_PARAMS = _build_params()

_CHUNK_SRC = {
    0: [0], 1: [0], 2: [0, 1], 3: [1], 4: [1],
    5: [1, 2], 6: [2], 7: [2, 3], 8: [3], 9: [3],
}
_SPLIT = {2: 10, 5: 4, 7: 14}
_ZT_LANE = {(2, 0): 2, (4, 1): 12, (5, 1): 0, (7, 2): 6}


def _body(x_hbm, prm_hbm, out_hbm, prm_v, pnl_v, out_v, sem_r, sem_w):
    wid = lax.axis_index("s") * 2 + lax.axis_index("c")
    pltpu.sync_copy(prm_hbm.at[wid], prm_v)
    lanesi = lax.iota(jnp.int32, 16)
    zero_i = lanesi * 0
    ixvecs = [zero_i + ix for ix in range(_N2)]
    lanesf = lanesi.astype(jnp.float32)
    onev = jnp.clip(lanesf + 1.0, 0.0, 1.0)
    mcv = {q: jnp.clip(float(s) - lanesf, 0.0, 1.0)
           for q, s in _SPLIT.items()}
    ztcv = {k: jnp.clip(lanesf - float(zl - 1), 0.0, 1.0)
            for k, zl in _ZT_LANE.items()}

    def load_scalars(s):
        rowA = prm_v[s, pl.ds(0, 16)]
        rowB = prm_v[s, pl.ds(16, 16)]
        vals = [rowA[0], rowA[1], rowA[2]]
        for ix in range(_N2):
            base = 4 + ix * 6
            src = rowA if base + 5 < 16 else rowB
            off = base if base + 5 < 16 else base - 16
            vals += [src[off + 0], src[off + 1], src[off + 2],
                     src[off + 3], src[off + 4], src[off + 5]]
        return tuple(vals)

    def sc_view(vals):
        sc = {"dzg": vals[0], "dy": vals[1], "eo1": vals[2],
              "szg": [], "sy": [], "dsy": [], "av": [], "m1eff": [],
              "flags": []}
        for ix in range(_N2):
            b = 3 + ix * 6
            sc["szg"].append(vals[b + 0])
            sc["sy"].append(vals[b + 1])
            sc["dsy"].append(vals[b + 2])
            sc["av"].append(vals[b + 3])
            sc["m1eff"].append(vals[b + 4])
            sc["flags"].append(vals[b + 5])
        return sc

    def reads(sc, slot, start):
        for ix in range(_N2):
            cp = pltpu.make_async_copy(
                x_hbm.at[sc["szg"][ix], pl.ds(sc["sy"][ix], 29), :],
                pnl_v.at[pl.ds((slot * 4 + ix) * 29, 29), :],
                sem_r.at[slot])
            if start:
                cp.start()
            else:
                cp.wait()

    def write_start(sc, slot):
        for E1 in (29, 15):
            @pl.when(sc["eo1"] == E1)
            def _(E1=E1):
                pltpu.make_async_copy(
                    out_v.at[slot, pl.ds(0, E1), :],
                    out_hbm.at[sc["dzg"], pl.ds(sc["dy"], E1), :],
                    sem_w.at[slot]).start()

    def write_wait(w, slot):
        dzg, dy, eo1 = w
        for E1 in (29, 15):
            @pl.when(eo1 == E1)
            def _(E1=E1):
                pltpu.make_async_copy(
                    out_v.at[slot, pl.ds(0, E1), :],
                    out_hbm.at[dzg, pl.ds(dy, E1), :],
                    sem_w.at[slot]).wait()

    def assemble(sc, slot):
        ztmul = {}
        for (q, ix), c in ztcv.items():
            zt_f = jnp.where((sc["flags"][ix] & 2) != 0, 1.0, 0.0)
            ztmul[(q, ix)] = onev - zt_f * c
        rbvec = [zero_i + ((slot * 4 + ix) * 29 + sc["dsy"][ix])
                 for ix in range(_N2)]

        @plsc.parallel_loop(0, sc["eo1"], unroll=2)
        def row_body(r):
            valid_f = [jnp.where(r < sc["m1eff"][ix], 1.0, 0.0)
                       for ix in range(_N2)]
            rvec = zero_i + r
            rowv = [rbvec[ix] + rvec for ix in range(_N2)]
            for q in range(10):
                pieces = []
                for ix in _CHUNK_SRC[q]:
                    off = sc["av"][ix] + (16 * q - _C0[ix])
                    v = plsc.load_gather(pnl_v, [rowv[ix], lanesi + off])
                    v = v * valid_f[ix]
                    if (q, ix) in ztmul:
                        v = v * ztmul[(q, ix)]
                    pieces.append(v)
                if len(pieces) == 2:
                    chunk = pieces[1] + (pieces[0] - pieces[1]) * mcv[q]
                else:
                    chunk = pieces[0]
                out_v[slot, r, pl.ds(16 * q, 16)] = chunk

    sc0_t = load_scalars(0)
    reads(sc_view(sc0_t), 0, True)
    dummy_w = (sc0_t[0] * 0, sc0_t[0] * 0, sc0_t[0] * 0 + 29)

    def slice_body(s, carry):
        sc_t, w1, w2 = carry
        slot = lax.rem(s, 2)
        sc = sc_view(sc_t)
        scn_t = load_scalars(s + 1)

        @pl.when(s < 59)
        def _():
            reads(sc_view(scn_t), 1 - slot, True)

        reads(sc, slot, False)

        @pl.when(s >= 2)
        def _():
            write_wait(w2, slot)

        assemble(sc, slot)
        write_start(sc, slot)
        return (scn_t, (sc["dzg"], sc["dy"], sc["eo1"]), w1)

    fin = lax.fori_loop(0, 60, slice_body, (sc0_t, dummy_w, dummy_w))
    write_wait(fin[2], 0)
    write_wait(fin[1], 1)


_run = pl.kernel(
    _body,
    out_type=jax.ShapeDtypeStruct((_B * _SHAPE[0], _SHAPE[1], _SHAPE[2]),
                                  jnp.float32),
    mesh=plsc.VectorSubcoreMesh(core_axis_name="c", subcore_axis_name="s"),
    compiler_params=pltpu.CompilerParams(use_tc_tiling_on_sc=False,
                                         needs_layout_passes=False),
    scratch_types=[
        pltpu.VMEM((61, 32), jnp.int32),
        pltpu.VMEM((232, 160), jnp.float32),
        pltpu.VMEM((2, 29, 160), jnp.float32),
        pltpu.SemaphoreType.DMA((2,)),
        pltpu.SemaphoreType.DMA((2,)),
    ],
)


def kernel(x):
    xf = x.reshape(_B * _SHAPE[0], _SHAPE[1], _SHAPE[2])
    prm = jnp.asarray(_PARAMS)
    out = _run(xf, prm)
    return out.reshape(_B, 1, *_SHAPE)
